# Initial kernel scaffold; baseline (speedup 1.0000x reference)
import jax
import jax.numpy as jnp
from jax.experimental import pallas as pl


def kernel(text, text_len, emb, W1, b1, W2, b2, W3, b3):
    raise NotImplementedError("write your pallas kernel here")


# speedup vs baseline: 9.6551x; 9.6551x over previous
import functools

import jax
import jax.numpy as jnp
from jax import lax
from jax.experimental import pallas as pl
from jax.experimental.pallas import tpu as pltpu
from jax.experimental.pallas import tpu_sc as plsc

NC, NS = 2, 16
NW = NC * NS
RING = 5


def _reduce_into(acc_ref, buf_ref, rows, add):
  d = acc_ref.shape[1]

  def row(i, _):
    for c in range(d // 16):
      sl = pl.ds(c * 16, 16)
      x = buf_ref[i, sl]
      if add:
        plsc.addupdate(acc_ref.at[i, sl], x)
      else:
        acc_ref[i, sl] = x
    return _

  lax.fori_loop(0, rows, row, None, unroll=2)


def _make_embbag(B, L, V, D):
  BW = B // NW
  mesh = plsc.VectorSubcoreMesh(core_axis_name="c", subcore_axis_name="s")

  @functools.partial(
      pl.kernel,
      out_type=jax.ShapeDtypeStruct((B, D), jnp.float32),
      mesh=mesh,
      scratch_types=[
          pltpu.VMEM((L, BW), jnp.int32),
          pltpu.VMEM((BW, D), jnp.float32),
          pltpu.VMEM((RING, BW, D), jnp.float32),
      ] + [pltpu.SemaphoreType.DMA] * RING,
  )
  def embbag(text_hbm, emb_hbm, out_hbm, idx_v, acc_v, bufs_v, *sems):
    wid = lax.axis_index("s") * NC + lax.axis_index("c")
    base = wid * BW
    pltpu.sync_copy(text_hbm.at[wid], idx_v)

    def start(j, slot):
      return pltpu.async_copy(
          emb_hbm.at[idx_v.at[j]], bufs_v.at[slot], sems[slot])

    def wait(slot):
      pltpu.make_async_copy(
          emb_hbm.at[idx_v.at[0]], bufs_v.at[slot], sems[slot]).wait()

    for r in range(RING):
      start(r, r)

    for r in range(RING):
      wait(r)
      _reduce_into(acc_v, bufs_v.at[r], BW, add=(r != 0))
      start(RING + r, r)

    def round_body(t, _):
      for r in range(RING):
        wait(r)
        _reduce_into(acc_v, bufs_v.at[r], BW, add=True)
        pltpu.async_copy(
            emb_hbm.at[idx_v.at[(t + 1) * RING + r]], bufs_v.at[r], sems[r])
      return _

    lax.fori_loop(1, L // RING - 1, round_body, None)

    for r in range(RING):
      wait(r)
      _reduce_into(acc_v, bufs_v.at[r], BW, add=True)

    pltpu.sync_copy(acc_v, out_hbm.at[pl.ds(base, BW)])

  return embbag


def _mlp_body(x_ref, len_ref, w1a_ref, w1b_ref, b1_ref, w2_ref, b2_ref,
              w3_ref, b3_ref, out_ref, *, inv_l):
  x = x_ref[...] * inv_l
  h = jnp.dot(x, w1a_ref[...], preferred_element_type=jnp.float32)
  h = h + len_ref[...] * w1b_ref[...] + b1_ref[...]
  h = jnp.maximum(h, 0.0)
  h = jnp.dot(h, w2_ref[...], preferred_element_type=jnp.float32) + b2_ref[...]
  h = jnp.maximum(h, 0.0)
  out_ref[...] = (
      jnp.dot(h, w3_ref[...], preferred_element_type=jnp.float32) + b3_ref[...])


def kernel(text, text_len, emb, W1, b1, W2, b2, W3, b3):
  B, L = text.shape
  V, D = emb.shape
  H = W1.shape[0]
  C = W3.shape[0]
  BW = B // NW

  text_r = text.astype(jnp.int32).reshape(NW, BW, L).transpose(0, 2, 1)
  xsum = _make_embbag(B, L, V, D)(text_r, emb)

  lens = text_len.astype(jnp.float32).reshape(B, 1)
  w1a = W1[:, :D].T
  w1b = W1[:, D].reshape(1, H)
  out = pl.pallas_call(
      functools.partial(_mlp_body, inv_l=1.0 / L),
      out_shape=jax.ShapeDtypeStruct((B, C), jnp.float32),
  )(xsum, lens, w1a, w1b, b1.reshape(1, H), W2.T, b2.reshape(1, H),
    W3.T, b3.reshape(1, C))
  return out


# speedup vs baseline: 11.8421x; 1.2265x over previous
import functools

import jax
import jax.numpy as jnp
from jax import lax
from jax.experimental import pallas as pl
from jax.experimental.pallas import tpu as pltpu
from jax.experimental.pallas import tpu_sc as plsc

NC, NS = 2, 16
NW = NC * NS
RING = 5


def _make_embbag(B, L, V, D):
  BW = B // NW
  mesh = plsc.VectorSubcoreMesh(core_axis_name="c", subcore_axis_name="s")

  @functools.partial(
      pl.kernel,
      out_type=jax.ShapeDtypeStruct((B, D), jnp.float32),
      mesh=mesh,
      scratch_types=[
          pltpu.VMEM((L, BW), jnp.int32),
          pltpu.VMEM((1, BW), jnp.int32),
          pltpu.VMEM((RING, BW, D), jnp.float32),
          pltpu.VMEM_SHARED((NS * BW, D), jnp.float32),
      ] + [pltpu.SemaphoreType.DMA] * (2 * RING),
  )
  def embbag(text_hbm, emb_hbm, out_hbm, idx_v, sidx_v, bufs_v, acc_s, *sems):
    gsems, ssems = sems[:RING], sems[RING:]
    cid = lax.axis_index("c")
    sid = lax.axis_index("s")
    wid = sid * NC + cid
    base = wid * BW
    accbase = sid * BW
    pltpu.sync_copy(text_hbm.at[wid], idx_v)
    for c in range(BW // 16):
      sidx_v[0, pl.ds(c * 16, 16)] = (
          lax.iota(jnp.int32, 16) + (accbase + c * 16))

    def start_g(j, slot):
      pltpu.async_copy(emb_hbm.at[idx_v.at[j]], bufs_v.at[slot], gsems[slot])

    def wait_g(slot):
      pltpu.make_async_copy(
          emb_hbm.at[idx_v.at[0]], bufs_v.at[slot], gsems[slot]).wait()

    def scatter_add(slot):
      pltpu.async_copy(
          bufs_v.at[slot], acc_s.at[sidx_v.at[0]], ssems[slot], add=True)
      pltpu.make_async_copy(
          bufs_v.at[slot], acc_s.at[sidx_v.at[0]], ssems[slot]).wait()

    for r in range(RING):
      start_g(r, r)

    wait_g(0)
    pltpu.sync_copy(bufs_v.at[0], acc_s.at[pl.ds(accbase, BW)])
    start_g(RING, 0)
    for r in range(1, RING):
      wait_g(r)
      scatter_add(r)
      start_g(RING + r, r)

    def round_body(t, _):
      for r in range(RING):
        wait_g(r)
        scatter_add(r)
        pltpu.async_copy(
            emb_hbm.at[idx_v.at[(t + 1) * RING + r]], bufs_v.at[r], gsems[r])
      return _

    lax.fori_loop(1, L // RING - 1, round_body, None)

    for r in range(RING):
      wait_g(r)
      scatter_add(r)

    pltpu.sync_copy(
        acc_s.at[pl.ds(accbase, BW)], out_hbm.at[pl.ds(base, BW)])

  return embbag


def _mlp_body(x_ref, len_ref, w1a_ref, w1b_ref, b1_ref, w2_ref, b2_ref,
              w3_ref, b3_ref, out_ref, *, inv_l):
  x = x_ref[...] * inv_l
  h = jnp.dot(x, w1a_ref[...], preferred_element_type=jnp.float32)
  h = h + len_ref[...] * w1b_ref[...] + b1_ref[...]
  h = jnp.maximum(h, 0.0)
  h = jnp.dot(h, w2_ref[...], preferred_element_type=jnp.float32) + b2_ref[...]
  h = jnp.maximum(h, 0.0)
  out_ref[...] = (
      jnp.dot(h, w3_ref[...], preferred_element_type=jnp.float32) + b3_ref[...])


def kernel(text, text_len, emb, W1, b1, W2, b2, W3, b3):
  B, L = text.shape
  V, D = emb.shape
  H = W1.shape[0]
  C = W3.shape[0]
  BW = B // NW

  text_r = text.astype(jnp.int32).reshape(NW, BW, L).transpose(0, 2, 1)
  xsum = _make_embbag(B, L, V, D)(text_r, emb)

  lens = text_len.astype(jnp.float32).reshape(B, 1)
  w1a = W1[:, :D].T
  w1b = W1[:, D].reshape(1, H)
  out = pl.pallas_call(
      functools.partial(_mlp_body, inv_l=1.0 / L),
      out_shape=jax.ShapeDtypeStruct((B, C), jnp.float32),
  )(xsum, lens, w1a, w1b, b1.reshape(1, H), W2.T, b2.reshape(1, H),
    W3.T, b3.reshape(1, C))
  return out
